# baseline (speedup 1.0000x reference)
import jax
import jax.numpy as jnp
from jax import lax
from jax.experimental import pallas as pl
from jax.experimental.pallas import tpu as pltpu
from jax.experimental.pallas import tpu_sc as plsc

N_NODES = 10000
N_EDGES = 320000
D = 128

NC = 2
NS = 16
NW = NC * NS

CHUNK = 128
NCHUNKS = N_EDGES // CHUNK
NITER = NCHUNKS // NW
NTAIL = NCHUNKS - NITER * NW

NB = N_NODES // 8


def _sc_body(x_hbm, src_hbm, msgs_hbm, src_v, rows_v, sem):
    c = lax.axis_index("c")
    s = lax.axis_index("s")
    wid = c * NS + s

    @pl.loop(0, NITER)
    def _(i):
        base = (i * NW + wid) * CHUNK
        pltpu.sync_copy(src_hbm.at[pl.ds(base, CHUNK)], src_v)
        pltpu.async_copy(x_hbm.at[src_v], rows_v, sem).wait()
        pltpu.sync_copy(rows_v, msgs_hbm.at[pl.ds(base, CHUNK)])

    @pl.when(wid < NTAIL)
    def _():
        base = (NITER * NW + wid) * CHUNK
        pltpu.sync_copy(src_hbm.at[pl.ds(base, CHUNK)], src_v)
        pltpu.async_copy(x_hbm.at[src_v], rows_v, sem).wait()
        pltpu.sync_copy(rows_v, msgs_hbm.at[pl.ds(base, CHUNK)])


@jax.jit
def _sc_gather(x, src):
    mesh = plsc.VectorSubcoreMesh(core_axis_name="c", subcore_axis_name="s")
    kern = pl.kernel(
        _sc_body,
        out_type=jax.ShapeDtypeStruct((N_EDGES, D), jnp.float32),
        mesh=mesh,
        scratch_types=[
            pltpu.VMEM((CHUNK,), jnp.int32),
            pltpu.VMEM((CHUNK, D), jnp.float32),
            pltpu.SemaphoreType.DMA,
        ],
    )
    return kern(x, src)


def _scat_body(dst_ref, msg_ref, agg_ref, deg_ref):
    pid = pl.program_id(0)

    @pl.when(pid == 0)
    def _():
        agg_ref[...] = jnp.zeros_like(agg_ref)
        deg_ref[...] = jnp.zeros_like(deg_ref)

    sub_iota = lax.broadcasted_iota(jnp.int32, (8, D), 0)
    for e in range(CHUNK):
        n = dst_ref[0, 0, e]
        blk = n // 8
        sub = n - blk * 8
        row = msg_ref[e:e + 1, :]
        m = (sub_iota == sub).astype(jnp.float32)
        agg_ref[pl.dslice(blk, 1)] += (m * row)[None]
        deg_ref[pl.dslice(blk, 1)] += m[None]


@jax.jit
def _tc_scatter(dst2d, msgs):
    return pl.pallas_call(
        _scat_body,
        grid=(NCHUNKS,),
        in_specs=[
            pl.BlockSpec((1, 1, CHUNK), lambda i: (i, 0, 0),
                         memory_space=pltpu.SMEM),
            pl.BlockSpec((CHUNK, D), lambda i: (i, 0)),
        ],
        out_specs=(
            pl.BlockSpec((NB, 8, D), lambda i: (0, 0, 0)),
            pl.BlockSpec((NB, 8, D), lambda i: (0, 0, 0)),
        ),
        out_shape=(
            jax.ShapeDtypeStruct((NB, 8, D), jnp.float32),
            jax.ShapeDtypeStruct((NB, 8, D), jnp.float32),
        ),
    )(dst2d, msgs)


def _tc_body(x_ref, p0_ref, d0_ref,
             wc_ref, bc_ref, w1_ref, b1_ref, w2_ref, b2_ref,
             w3_ref, b3_ref, out_ref):
    agg = p0_ref[...]
    deg = d0_ref[:, 0:1]
    deg = jnp.maximum(deg, 1.0)
    agg = agg / deg
    h = agg @ wc_ref[...] + bc_ref[...]
    h = jnp.maximum(h, 0.0)
    z = h + x_ref[...]
    z = z @ w1_ref[...] + b1_ref[...]
    z = jnp.where(z >= 0.0, z, 0.01 * z)
    z = z @ w2_ref[...] + b2_ref[...]
    z = jnp.where(z >= 0.0, z, 0.01 * z)
    z = z @ w3_ref[...] + b3_ref[...]
    out_ref[...] = jnp.maximum(z, 0.0) + jnp.log1p(jnp.exp(-jnp.abs(z)))


@jax.jit
def _tc_head(x, p0, d0, W_conv, b_conv, W1, b1, W2, b2, W3, b3):
    blk = 1000
    grid = (N_NODES // blk,)
    row_spec = pl.BlockSpec((blk, D), lambda i: (i, 0))

    def w_spec(shape):
        return pl.BlockSpec(shape, lambda i: tuple(0 for _ in shape))

    return pl.pallas_call(
        _tc_body,
        grid=grid,
        in_specs=[
            row_spec, row_spec, row_spec,
            w_spec((D, D)), w_spec((1, D)),
            w_spec((D, D)), w_spec((1, D)),
            w_spec((D, D)), w_spec((1, D)),
            w_spec((D, 1)), w_spec((1, 1)),
        ],
        out_specs=pl.BlockSpec((blk, 1), lambda i: (i, 0)),
        out_shape=jax.ShapeDtypeStruct((N_NODES, 1), jnp.float32),
    )(x, p0, d0, W_conv, b_conv.reshape(1, D), W1, b1.reshape(1, D),
      W2, b2.reshape(1, D), W3, b3.reshape(1, 1))


def kernel(x, edge_index, W_conv, b_conv, W1, b1, W2, b2, W3, b3):
    ei = edge_index.astype(jnp.int32)
    msgs = _sc_gather(x, ei[0])
    agg3, deg3 = _tc_scatter(ei[1].reshape(NCHUNKS, 1, CHUNK), msgs)
    out = _tc_head(x, agg3.reshape(N_NODES, D), deg3.reshape(N_NODES, D),
                   W_conv, b_conv, W1, b1, W2, b2, W3, b3)
    return out.reshape(-1) + 1e-20
